# baseline (speedup 1.0000x reference)
import jax
import jax.numpy as jnp
from jax.experimental import pallas as pl
from jax.experimental.pallas import tpu as pltpu

BATCH = 1024
IN_FEATURES = 128
OUT_FEATURES = 128
GRID_SIZE = 8
MIN_VALUE = -2.0
MAX_VALUE = 2.0

BLOCK_B = 256


def _kan_kernel(x_ref, basis_ref, bias_ref, out_ref):
    x = x_ref[:]
    scaled = (jnp.clip(x, MIN_VALUE, MAX_VALUE) - MIN_VALUE) * (
        (GRID_SIZE - 1) / (MAX_VALUE - MIN_VALUE)
    )
    acc = jnp.broadcast_to(bias_ref[:], out_ref.shape)
    for g in range(GRID_SIZE):
        w = jnp.maximum(1.0 - jnp.abs(scaled - float(g)), 0.0).astype(jnp.bfloat16)
        acc = acc + jnp.dot(w, basis_ref[g], preferred_element_type=jnp.float32)
    out_ref[:] = acc


def kernel(inputs, basis, bias):
    basis_t = jnp.transpose(basis.astype(jnp.bfloat16), (2, 1, 0))
    bias2d = bias.reshape(1, OUT_FEATURES)
    grid = (BATCH // BLOCK_B,)
    return pl.pallas_call(
        _kan_kernel,
        grid=grid,
        in_specs=[
            pl.BlockSpec((BLOCK_B, IN_FEATURES), lambda i: (i, 0)),
            pl.BlockSpec((GRID_SIZE, IN_FEATURES, OUT_FEATURES), lambda i: (0, 0, 0)),
            pl.BlockSpec((1, OUT_FEATURES), lambda i: (0, 0)),
        ],
        out_specs=pl.BlockSpec((BLOCK_B, OUT_FEATURES), lambda i: (i, 0)),
        out_shape=jax.ShapeDtypeStruct((BATCH, OUT_FEATURES), jnp.float32),
    )(inputs, basis_t, bias2d)


# speedup vs baseline: 1.2323x; 1.2323x over previous
import jax
import jax.numpy as jnp
from jax.experimental import pallas as pl
from jax.experimental.pallas import tpu as pltpu

BATCH = 1024
IN_FEATURES = 128
OUT_FEATURES = 128
GRID_SIZE = 8
MIN_VALUE = -2.0
MAX_VALUE = 2.0

BLOCK_B = 512


def _kan_kernel(x_ref, basis_ref, bias_ref, out_ref):
    x = x_ref[:]
    scaled = (jnp.clip(x, MIN_VALUE, MAX_VALUE) - MIN_VALUE) * (
        (GRID_SIZE - 1) / (MAX_VALUE - MIN_VALUE)
    )
    acc = jnp.broadcast_to(bias_ref[:], out_ref.shape)
    for g in range(GRID_SIZE):
        w = jnp.maximum(1.0 - jnp.abs(scaled - float(g)), 0.0).astype(jnp.bfloat16)
        acc = acc + jnp.dot(w, basis_ref[g], preferred_element_type=jnp.float32)
    out_ref[:] = acc


def kernel(inputs, basis, bias):
    basis_t = jnp.transpose(basis, (2, 1, 0)).astype(jnp.bfloat16)
    bias2d = bias.reshape(1, OUT_FEATURES)
    grid = (BATCH // BLOCK_B,)
    return pl.pallas_call(
        _kan_kernel,
        grid=grid,
        in_specs=[
            pl.BlockSpec((BLOCK_B, IN_FEATURES), lambda i: (i, 0)),
            pl.BlockSpec((GRID_SIZE, IN_FEATURES, OUT_FEATURES), lambda i: (0, 0, 0)),
            pl.BlockSpec((1, OUT_FEATURES), lambda i: (0, 0)),
        ],
        out_specs=pl.BlockSpec((BLOCK_B, OUT_FEATURES), lambda i: (i, 0)),
        out_shape=jax.ShapeDtypeStruct((BATCH, OUT_FEATURES), jnp.float32),
    )(inputs, basis_t, bias2d)


# speedup vs baseline: 1.2427x; 1.0085x over previous
import jax
import jax.numpy as jnp
from jax.experimental import pallas as pl
from jax.experimental.pallas import tpu as pltpu

BATCH = 1024
IN_FEATURES = 128
OUT_FEATURES = 128
GRID_SIZE = 8
MIN_VALUE = -2.0
MAX_VALUE = 2.0

BLOCK_B = 1024


def _kan_kernel(x_ref, basis_ref, bias_ref, out_ref):
    x = x_ref[:]
    scaled = (jnp.clip(x, MIN_VALUE, MAX_VALUE) - MIN_VALUE) * (
        (GRID_SIZE - 1) / (MAX_VALUE - MIN_VALUE)
    )
    acc = jnp.broadcast_to(bias_ref[:], out_ref.shape)
    for g in range(GRID_SIZE):
        w = jnp.maximum(1.0 - jnp.abs(scaled - float(g)), 0.0).astype(jnp.bfloat16)
        acc = acc + jnp.dot(w, basis_ref[g], preferred_element_type=jnp.float32)
    out_ref[:] = acc


def kernel(inputs, basis, bias):
    basis_t = jnp.transpose(basis, (2, 1, 0)).astype(jnp.bfloat16)
    bias2d = bias.reshape(1, OUT_FEATURES)
    grid = (BATCH // BLOCK_B,)
    return pl.pallas_call(
        _kan_kernel,
        grid=grid,
        in_specs=[
            pl.BlockSpec((BLOCK_B, IN_FEATURES), lambda i: (i, 0)),
            pl.BlockSpec((GRID_SIZE, IN_FEATURES, OUT_FEATURES), lambda i: (0, 0, 0)),
            pl.BlockSpec((1, OUT_FEATURES), lambda i: (0, 0)),
        ],
        out_specs=pl.BlockSpec((BLOCK_B, OUT_FEATURES), lambda i: (i, 0)),
        out_shape=jax.ShapeDtypeStruct((BATCH, OUT_FEATURES), jnp.float32),
    )(inputs, basis_t, bias2d)
